# Initial kernel scaffold; baseline (speedup 1.0000x reference)
import jax
import jax.numpy as jnp
from jax.experimental import pallas as pl


def kernel(R):
    raise NotImplementedError("write your pallas kernel here")


# speedup vs baseline: 6.4905x; 6.4905x over previous
import functools

import jax
import jax.numpy as jnp
from jax import lax
from jax.experimental import pallas as pl

K = 64
EXCLUDE = 2
R_ON = 8.0
R_CUT = 10.0
R0 = 4.0
DELTA = 0.2
WALL_SCALE = 10.0
MASK_D2 = 1.0e18

_RB = 256


def _pair_energy(r):
    x = (R0 - r) / (DELTA + 1e-12)
    sp = jnp.maximum(x, 0.0) + jnp.log1p(jnp.exp(-jnp.abs(x)))
    t = jnp.clip((R_CUT - r) / (R_CUT - R_ON), 0.0, 1.0)
    sw = t * t * (3.0 - 2.0 * t)
    return (WALL_SCALE * sp) * sw


def _body(r_ref, rt_ref, o_ref):
    blk = pl.program_id(1)
    rows = r_ref[0]
    cols = rt_ref[0]
    RB = rows.shape[0]
    L = cols.shape[1]

    xi = rows[:, 0:1]
    yi = rows[:, 1:2]
    zi = rows[:, 2:3]
    xj = cols[0:1, :]
    yj = cols[1:2, :]
    zj = cols[2:3, :]
    dx = xi - xj
    dy = yi - yj
    dz = zi - zj
    d2 = dx * dx + dy * dy + dz * dz

    i_idx = blk * RB + lax.broadcasted_iota(jnp.int32, (RB, 1), 0)
    j_idx = lax.broadcasted_iota(jnp.int32, (1, L), 1)
    excl = jnp.abs(i_idx - j_idx) <= EXCLUDE
    d2m = jnp.where(excl, MASK_D2, d2)

    v = lax.bitcast_convert_type(d2m, jnp.int32)

    def bit_step(i, p):
        bit = lax.shift_left(jnp.int32(1), jnp.int32(30) - i)
        cand = p | bit
        cnt = jnp.sum((v < cand).astype(jnp.int32), axis=1, keepdims=True)
        return jnp.where(cnt <= K - 1, cand, p)

    p0 = jnp.zeros((RB, 1), dtype=jnp.int32)
    p = lax.fori_loop(0, 31, bit_step, p0)

    tau2 = lax.bitcast_convert_type(p, jnp.float32)
    sel = v < p
    c = jnp.sum(sel.astype(jnp.float32), axis=1, keepdims=True)

    r = jnp.sqrt(d2m + 1e-12)
    f = _pair_energy(r)
    s = jnp.sum(jnp.where(sel, f, 0.0), axis=1, keepdims=True)

    f_tau = _pair_energy(jnp.sqrt(tau2 + 1e-12))
    row_total = s + (K - c) * f_tau
    o_ref[0, 0] = jnp.full((8, 128), jnp.sum(row_total), jnp.float32)


@jax.jit
def kernel(R):
    B, L, _ = R.shape
    RT = jnp.swapaxes(R, 1, 2)
    nblk = L // _RB
    out = pl.pallas_call(
        _body,
        grid=(B, nblk),
        in_specs=[
            pl.BlockSpec((1, _RB, 3), lambda b, k: (b, k, 0)),
            pl.BlockSpec((1, 3, L), lambda b, k: (b, 0, 0)),
        ],
        out_specs=pl.BlockSpec((1, 1, 8, 128), lambda b, k: (b, k, 0, 0)),
        out_shape=jax.ShapeDtypeStruct((B, nblk, 8, 128), jnp.float32),
    )(R, RT)
    return jnp.sum(out[:, :, 0, 0], axis=1)


# speedup vs baseline: 8.6481x; 1.3324x over previous
import functools

import jax
import jax.numpy as jnp
from jax import lax
from jax.experimental import pallas as pl

K = 64
EXCLUDE = 2
R_ON = 8.0
R_CUT = 10.0
R0 = 4.0
DELTA = 0.2
WALL_SCALE = 10.0
MASK_D2 = 1.0e18

_RB = 256


def _pair_energy(r):
    x = (R0 - r) / (DELTA + 1e-12)
    sp = jnp.maximum(x, 0.0) + jnp.log1p(jnp.exp(-jnp.abs(x)))
    t = jnp.clip((R_CUT - r) / (R_CUT - R_ON), 0.0, 1.0)
    sw = t * t * (3.0 - 2.0 * t)
    return (WALL_SCALE * sp) * sw


def _body(r_ref, rt_ref, o_ref):
    blk = pl.program_id(1)
    rows = r_ref[0]
    cols = rt_ref[0]
    RB = rows.shape[0]
    L = cols.shape[1]

    xi = rows[:, 0:1]
    yi = rows[:, 1:2]
    zi = rows[:, 2:3]
    xj = cols[0:1, :]
    yj = cols[1:2, :]
    zj = cols[2:3, :]
    dx = xi - xj
    dy = yi - yj
    dz = zi - zj
    d2 = dx * dx + dy * dy + dz * dz

    i_idx = blk * RB + lax.broadcasted_iota(jnp.int32, (RB, 1), 0)
    j_idx = lax.broadcasted_iota(jnp.int32, (1, L), 1)
    excl = jnp.abs(i_idx - j_idx) <= EXCLUDE
    d2m = jnp.where(excl, MASK_D2, d2)

    v = lax.shift_right_logical(lax.bitcast_convert_type(d2m, jnp.int32), 15)

    def bit_step(i, q):
        bit = lax.shift_left(jnp.int32(1), jnp.int32(15) - i)
        cand = q | bit
        cnt = jnp.sum((v < cand).astype(jnp.int32), axis=1, keepdims=True)
        return jnp.where(cnt <= K - 1, cand, q)

    q0 = jnp.zeros((RB, 1), dtype=jnp.int32)
    q = lax.fori_loop(0, 16, bit_step, q0)

    tau2 = lax.bitcast_convert_type(
        lax.shift_left(q, 15) + jnp.int32(0x4000), jnp.float32)
    sel = v < q
    c = jnp.sum(sel.astype(jnp.float32), axis=1, keepdims=True)

    r = jnp.sqrt(d2m + 1e-12)
    f = _pair_energy(r)
    s = jnp.sum(jnp.where(sel, f, 0.0), axis=1, keepdims=True)

    f_tau = _pair_energy(jnp.sqrt(tau2 + 1e-12))
    row_total = s + (K - c) * f_tau
    o_ref[0, 0] = jnp.full((8, 128), jnp.sum(row_total), jnp.float32)


@jax.jit
def kernel(R):
    B, L, _ = R.shape
    RT = jnp.swapaxes(R, 1, 2)
    nblk = L // _RB
    out = pl.pallas_call(
        _body,
        grid=(B, nblk),
        in_specs=[
            pl.BlockSpec((1, _RB, 3), lambda b, k: (b, k, 0)),
            pl.BlockSpec((1, 3, L), lambda b, k: (b, 0, 0)),
        ],
        out_specs=pl.BlockSpec((1, 1, 8, 128), lambda b, k: (b, k, 0, 0)),
        out_shape=jax.ShapeDtypeStruct((B, nblk, 8, 128), jnp.float32),
    )(R, RT)
    return jnp.sum(out[:, :, 0, 0], axis=1)
